# Initial kernel scaffold; baseline (speedup 1.0000x reference)
import jax
import jax.numpy as jnp
from jax.experimental import pallas as pl


def kernel(x, edge_index, batch, bn_g, bn_b, conv_W, conv_b, Wf, bf, gf, bbf, gh, bh, Wc, bc):
    raise NotImplementedError("write your pallas kernel here")


# speedup vs baseline: 15.8623x; 15.8623x over previous
import functools

import jax
import jax.numpy as jnp
from jax import lax
from jax.experimental import pallas as pl
from jax.experimental.pallas import tpu as pltpu
from jax.experimental.pallas import tpu_sc as plsc

N = 10000
E = 320000
D = 128
G = 128
C = 10

NC = 2
NS = 16
NW = NC * NS

CH = 80
EPT = E // NW
NCHUNK = EPT // CH
ZROWS = 32
WB = 640
WSTRIDE = 624

def _sc_deg_body(dst_hbm, out_hbm, dst_v, ones_v, z_v, acc_sh):
    cid = lax.axis_index("c")
    sid = lax.axis_index("s")
    g = cid * NS + sid

    pltpu.sync_copy(dst_hbm.at[g], dst_v)

    def fill(i, _):
        z_v[pl.ds(i * 16, 16)] = jnp.zeros((16,), jnp.float32)
        return 0

    lax.fori_loop(0, WB // 16, fill, 0)
    for j in range(CH // 16):
        ones_v[pl.ds(j * 16, 16)] = jnp.ones((16,), jnp.float32)

    pltpu.sync_copy(z_v, acc_sh.at[pl.ds(sid * WSTRIDE, WB)])
    plsc.subcore_barrier()

    def body(j, _):
        pltpu.sync_copy(ones_v, acc_sh.at[dst_v.at[j]], add=True)
        return 0

    lax.fori_loop(0, NCHUNK, body, 0)
    plsc.subcore_barrier()
    pltpu.sync_copy(acc_sh.at[pl.ds(sid * WSTRIDE, WB)], z_v)
    pltpu.sync_copy(z_v, out_hbm.at[pl.ds(cid * N + sid * WSTRIDE, WB)])


def _sc_edges_body(h_hbm, src_hbm, dst_hbm, out_hbm, src_v, dst_v, rows_v,
                   z_v, acc_sh, sem):
    cid = lax.axis_index("c")
    sid = lax.axis_index("s")
    g = cid * NS + sid

    pltpu.sync_copy(src_hbm.at[g], src_v)
    pltpu.sync_copy(dst_hbm.at[g], dst_v)

    def fill(i, _):
        for j in range(D // 16):
            z_v[i, pl.ds(j * 16, 16)] = jnp.zeros((16,), jnp.float32)
        return 0

    lax.fori_loop(0, ZROWS, fill, 0)
    for k in range(WB // ZROWS):
        pltpu.sync_copy(z_v, acc_sh.at[pl.ds(sid * WSTRIDE + k * ZROWS, ZROWS)])
    plsc.subcore_barrier()

    def body(j, _):
        pltpu.async_copy(h_hbm.at[src_v.at[j]], rows_v, sem).wait()
        pltpu.sync_copy(rows_v, acc_sh.at[dst_v.at[j]], add=True)
        return 0

    lax.fori_loop(0, NCHUNK, body, 0)
    plsc.subcore_barrier()
    for k in range(WB // ZROWS):
        pltpu.sync_copy(
            acc_sh.at[pl.ds(sid * WSTRIDE + k * ZROWS, ZROWS)], z_v)
        pltpu.sync_copy(
            z_v, out_hbm.at[cid, pl.ds(sid * WSTRIDE + k * ZROWS, ZROWS)])


@functools.cache
def _sc_kernels():
    mesh = plsc.VectorSubcoreMesh(core_axis_name="c", subcore_axis_name="s",
                                  num_cores=NC, num_subcores=NS)
    sc_deg = pl.kernel(
        _sc_deg_body,
        out_type=jax.ShapeDtypeStruct((NC * N,), jnp.float32),
        mesh=mesh,
        scratch_types=[
            pltpu.VMEM((NCHUNK, CH), jnp.int32),
            pltpu.VMEM((CH,), jnp.float32),
            pltpu.VMEM((WB,), jnp.float32),
            pltpu.VMEM_SHARED((N,), jnp.float32),
        ],
    )
    sc_edges = pl.kernel(
        _sc_edges_body,
        out_type=jax.ShapeDtypeStruct((NC, N, D), jnp.float32),
        mesh=mesh,
        scratch_types=[
            pltpu.VMEM((NCHUNK, CH), jnp.int32),
            pltpu.VMEM((NCHUNK, CH), jnp.int32),
            pltpu.VMEM((CH, D), jnp.float32),
            pltpu.VMEM((ZROWS, D), jnp.float32),
            pltpu.VMEM_SHARED((N, D), jnp.float32),
            pltpu.SemaphoreType.DMA,
        ],
    )
    return sc_deg, sc_edges


def _bn(x, g, b):
    m = jnp.mean(x, axis=0)
    v = jnp.mean((x - m) * (x - m), axis=0)
    return (x - m) * lax.rsqrt(v + 1e-5) * g + b


def _tc_first(x_ref, deg_ref, g_ref, b_ref, w_ref, hs_ref, dinv_ref):
    deg = deg_ref[:, 0:1] + deg_ref[:, 1:2] + 1.0
    dinv = lax.rsqrt(deg)
    dinv_ref[...] = dinv
    xn = _bn(x_ref[...], g_ref[...], b_ref[...])
    h = jnp.dot(xn, w_ref[...], preferred_element_type=jnp.float32)
    hs_ref[...] = dinv * h


def _tc_mid(x_ref, acc_ref, hs_ref, dinv_ref, cb_ref, g_ref, b_ref, w_ref,
            x_out_ref, hs_out_ref):
    dinv = dinv_ref[...]
    agg = acc_ref[0] + acc_ref[1] + hs_ref[...]
    x = x_ref[...] + jnp.maximum(dinv * agg + cb_ref[...], 0.0)
    x_out_ref[...] = x
    xn = _bn(x, g_ref[...], b_ref[...])
    h = jnp.dot(xn, w_ref[...], preferred_element_type=jnp.float32)
    hs_out_ref[...] = dinv * h


def _tc_head(x_ref, acc_ref, hs_ref, dinv_ref, cb_ref, batch_ref, wf_ref,
             bf_ref, gf_ref, bbf_ref, gh_ref, bh_ref, wc_ref, bc_ref,
             out_ref):
    dinv = dinv_ref[...]
    agg = acc_ref[0] + acc_ref[1] + hs_ref[...]
    x = x_ref[...] + jnp.maximum(dinv * agg + cb_ref[...], 0.0)

    gids = lax.broadcasted_iota(jnp.int32, (G, N), 0)
    onehot = (gids == batch_ref[...]).astype(jnp.float32)
    sums = jnp.dot(onehot, x, preferred_element_type=jnp.float32)
    cnt = jnp.sum(onehot, axis=1, keepdims=True)
    xg = sums / jnp.maximum(cnt, 1.0)

    h = _bn(xg, gf_ref[...], bbf_ref[...])
    h = jnp.maximum(jnp.dot(h, wf_ref[...], preferred_element_type=jnp.float32)
                    + bf_ref[...], 0.0)
    h = _bn(h, gh_ref[...], bh_ref[...])
    logits = jnp.dot(h, wc_ref[...], preferred_element_type=jnp.float32) \
        + bc_ref[...]
    m = jnp.max(logits, axis=1, keepdims=True)
    lse = jnp.log(jnp.sum(jnp.exp(logits - m), axis=1, keepdims=True)) + m
    out_ref[...] = logits - lse


def kernel(x, edge_index, batch, bn_g, bn_b, conv_W, conv_b, Wf, bf, gf, bbf,
           gh, bh, Wc, bc):
    f32 = jnp.float32
    src2d = edge_index[0].reshape(NW, NCHUNK, CH)
    dst2d = edge_index[1].reshape(NW, NCHUNK, CH)

    sc_deg, sc_edges = _sc_kernels()
    deg2 = sc_deg(dst2d).reshape(NC, N).T

    tc_first = pl.pallas_call(
        _tc_first,
        out_shape=[jax.ShapeDtypeStruct((N, D), f32),
                   jax.ShapeDtypeStruct((N, 1), f32)],
    )
    tc_mid = pl.pallas_call(
        _tc_mid,
        out_shape=[jax.ShapeDtypeStruct((N, D), f32),
                   jax.ShapeDtypeStruct((N, D), f32)],
    )
    tc_head = pl.pallas_call(
        _tc_head,
        out_shape=jax.ShapeDtypeStruct((G, G), f32),
    )

    hs, dinv = tc_first(x, deg2, bn_g[0].reshape(1, D), bn_b[0].reshape(1, D),
                        conv_W[0])
    acc = sc_edges(hs, src2d, dst2d)
    hs_prev = hs
    x_cur = x
    for layer in (1, 2):
        x_cur, hs = tc_mid(x_cur, acc, hs_prev,
                           dinv, conv_b[layer - 1].reshape(1, D),
                           bn_g[layer].reshape(1, D),
                           bn_b[layer].reshape(1, D), conv_W[layer])
        acc = sc_edges(hs, src2d, dst2d)
        hs_prev = hs

    wc_pad = jnp.zeros((D, G), f32).at[:, :C].set(Wc)
    bc_pad = jnp.full((G,), -1e30, f32).at[:C].set(bc)
    out = tc_head(x_cur, acc, hs_prev, dinv, conv_b[2].reshape(1, D),
                  batch.reshape(1, N), Wf, bf.reshape(1, D),
                  gf.reshape(1, D), bbf.reshape(1, D), gh.reshape(1, D),
                  bh.reshape(1, D), wc_pad, bc_pad.reshape(1, G))
    return out[:, :C]


# speedup vs baseline: 24.2449x; 1.5285x over previous
import functools

import jax
import jax.numpy as jnp
from jax import lax
from jax.experimental import pallas as pl
from jax.experimental.pallas import tpu as pltpu
from jax.experimental.pallas import tpu_sc as plsc

N = 10000
E = 320000
D = 128
G = 128
C = 10

NC = 2
NS = 16
NW = NC * NS

CH = 80
EPT = E // NW
NCHUNK = EPT // CH
IBLK = 25
NBLK = NCHUNK // IBLK
ZROWS = 32
WB = 640
WSTRIDE = 624

def _sc_deg_body(dst_hbm, out_hbm, dst_v, ones_v, z_v, acc_sh):
    cid = lax.axis_index("c")
    sid = lax.axis_index("s")
    g = cid * NS + sid

    for i in range(NBLK):
        pltpu.sync_copy(dst_hbm.at[g, i], dst_v.at[i])

    def fill(i, _):
        z_v[pl.ds(i * 16, 16)] = jnp.zeros((16,), jnp.float32)
        return 0

    lax.fori_loop(0, WB // 16, fill, 0)
    for j in range(CH // 16):
        ones_v[pl.ds(j * 16, 16)] = jnp.ones((16,), jnp.float32)

    pltpu.sync_copy(z_v, acc_sh.at[pl.ds(sid * WSTRIDE, WB)])
    plsc.subcore_barrier()

    def body(j, _):
        pltpu.sync_copy(ones_v, acc_sh.at[dst_v.at[j // IBLK, lax.rem(j, IBLK)]],
                        add=True)
        return 0

    lax.fori_loop(0, NCHUNK, body, 0)
    plsc.subcore_barrier()
    pltpu.sync_copy(acc_sh.at[pl.ds(sid * WSTRIDE, WB)], z_v)
    pltpu.sync_copy(z_v, out_hbm.at[pl.ds(cid * N + sid * WSTRIDE, WB)])


def _sc_edges_body(h_hbm, src_hbm, dst_hbm, out_hbm, src_v, dst_v, rows_v,
                   z_v, acc_sh, sem):
    cid = lax.axis_index("c")
    sid = lax.axis_index("s")
    g = cid * NS + sid

    def fill(i, _):
        for j in range(D // 16):
            z_v[i, pl.ds(j * 16, 16)] = jnp.zeros((16,), jnp.float32)
        return 0

    lax.fori_loop(0, ZROWS, fill, 0)
    for k in range(WB // ZROWS):
        pltpu.sync_copy(z_v, acc_sh.at[pl.ds(sid * WSTRIDE + k * ZROWS, ZROWS)])

    pltpu.sync_copy(src_hbm.at[g, 0], src_v.at[0])
    pltpu.sync_copy(dst_hbm.at[g, 0], dst_v.at[0])
    plsc.subcore_barrier()

    pltpu.async_copy(h_hbm.at[src_v.at[0, 0]], rows_v.at[0], sem)

    def body(k, _):
        b = lax.rem(k, 2)
        blk = k // IBLK
        jj = lax.rem(k, IBLK)
        sl = lax.rem(blk, 2)

        @pl.when(jnp.logical_and(jj == 0, blk + 1 < NBLK))
        def _():
            nsl = lax.rem(blk + 1, 2)
            pltpu.sync_copy(src_hbm.at[g, blk + 1], src_v.at[nsl])
            pltpu.sync_copy(dst_hbm.at[g, blk + 1], dst_v.at[nsl])

        @pl.when(k + 1 < NCHUNK)
        def _():
            k1 = k + 1
            bl1 = k1 // IBLK
            pltpu.async_copy(
                h_hbm.at[src_v.at[lax.rem(bl1, 2), lax.rem(k1, IBLK)]],
                rows_v.at[lax.rem(k1, 2)], sem)

        pltpu.make_async_copy(
            h_hbm.at[src_v.at[sl, jj]], rows_v.at[b], sem).wait()
        pltpu.sync_copy(rows_v.at[b], acc_sh.at[dst_v.at[sl, jj]], add=True)
        return 0

    lax.fori_loop(0, NCHUNK, body, 0)
    plsc.subcore_barrier()
    for k in range(WB // ZROWS):
        pltpu.sync_copy(
            acc_sh.at[pl.ds(sid * WSTRIDE + k * ZROWS, ZROWS)], z_v)
        pltpu.sync_copy(
            z_v, out_hbm.at[cid, pl.ds(sid * WSTRIDE + k * ZROWS, ZROWS)])


@functools.cache
def _sc_kernels():
    mesh = plsc.VectorSubcoreMesh(core_axis_name="c", subcore_axis_name="s",
                                  num_cores=NC, num_subcores=NS)
    sc_deg = pl.kernel(
        _sc_deg_body,
        out_type=jax.ShapeDtypeStruct((NC * N,), jnp.float32),
        mesh=mesh,
        scratch_types=[
            pltpu.VMEM((NBLK, IBLK, CH), jnp.int32),
            pltpu.VMEM((CH,), jnp.float32),
            pltpu.VMEM((WB,), jnp.float32),
            pltpu.VMEM_SHARED((N,), jnp.float32),
        ],
    )
    sc_edges = pl.kernel(
        _sc_edges_body,
        out_type=jax.ShapeDtypeStruct((NC, N, D), jnp.float32),
        mesh=mesh,
        scratch_types=[
            pltpu.VMEM((2, IBLK, CH), jnp.int32),
            pltpu.VMEM((2, IBLK, CH), jnp.int32),
            pltpu.VMEM((2, CH, D), jnp.float32),
            pltpu.VMEM((ZROWS, D), jnp.float32),
            pltpu.VMEM_SHARED((N, D), jnp.float32),
            pltpu.SemaphoreType.DMA,
        ],
    )
    return sc_deg, sc_edges


def _bn(x, g, b):
    m = jnp.mean(x, axis=0)
    v = jnp.mean((x - m) * (x - m), axis=0)
    return (x - m) * lax.rsqrt(v + 1e-5) * g + b


def _tc_first(x_ref, deg_ref, g_ref, b_ref, w_ref, hs_ref, dinv_ref):
    deg = deg_ref[:, 0:1] + deg_ref[:, 1:2] + 1.0
    dinv = lax.rsqrt(deg)
    dinv_ref[...] = dinv
    xn = _bn(x_ref[...], g_ref[...], b_ref[...])
    h = jnp.dot(xn, w_ref[...], preferred_element_type=jnp.float32)
    hs_ref[...] = dinv * h


def _tc_mid(x_ref, acc_ref, hs_ref, dinv_ref, cb_ref, g_ref, b_ref, w_ref,
            x_out_ref, hs_out_ref):
    dinv = dinv_ref[...]
    agg = acc_ref[0] + acc_ref[1] + hs_ref[...]
    x = x_ref[...] + jnp.maximum(dinv * agg + cb_ref[...], 0.0)
    x_out_ref[...] = x
    xn = _bn(x, g_ref[...], b_ref[...])
    h = jnp.dot(xn, w_ref[...], preferred_element_type=jnp.float32)
    hs_out_ref[...] = dinv * h


def _tc_head(x_ref, acc_ref, hs_ref, dinv_ref, cb_ref, batch_ref, wf_ref,
             bf_ref, gf_ref, bbf_ref, gh_ref, bh_ref, wc_ref, bc_ref,
             out_ref):
    dinv = dinv_ref[...]
    agg = acc_ref[0] + acc_ref[1] + hs_ref[...]
    x = x_ref[...] + jnp.maximum(dinv * agg + cb_ref[...], 0.0)

    gids = lax.broadcasted_iota(jnp.int32, (G, N), 0)
    onehot = (gids == batch_ref[...]).astype(jnp.float32)
    sums = jnp.dot(onehot, x, preferred_element_type=jnp.float32)
    cnt = jnp.sum(onehot, axis=1, keepdims=True)
    xg = sums / jnp.maximum(cnt, 1.0)

    h = _bn(xg, gf_ref[...], bbf_ref[...])
    h = jnp.maximum(jnp.dot(h, wf_ref[...], preferred_element_type=jnp.float32)
                    + bf_ref[...], 0.0)
    h = _bn(h, gh_ref[...], bh_ref[...])
    logits = jnp.dot(h, wc_ref[...], preferred_element_type=jnp.float32) \
        + bc_ref[...]
    m = jnp.max(logits, axis=1, keepdims=True)
    lse = jnp.log(jnp.sum(jnp.exp(logits - m), axis=1, keepdims=True)) + m
    out_ref[...] = logits - lse


def kernel(x, edge_index, batch, bn_g, bn_b, conv_W, conv_b, Wf, bf, gf, bbf,
           gh, bh, Wc, bc):
    f32 = jnp.float32
    src2d = edge_index[0].reshape(NW, NBLK, IBLK, CH)
    dst2d = edge_index[1].reshape(NW, NBLK, IBLK, CH)

    sc_deg, sc_edges = _sc_kernels()
    deg2 = sc_deg(dst2d).reshape(NC, N).T

    tc_first = pl.pallas_call(
        _tc_first,
        out_shape=[jax.ShapeDtypeStruct((N, D), f32),
                   jax.ShapeDtypeStruct((N, 1), f32)],
    )
    tc_mid = pl.pallas_call(
        _tc_mid,
        out_shape=[jax.ShapeDtypeStruct((N, D), f32),
                   jax.ShapeDtypeStruct((N, D), f32)],
    )
    tc_head = pl.pallas_call(
        _tc_head,
        out_shape=jax.ShapeDtypeStruct((G, G), f32),
    )

    hs, dinv = tc_first(x, deg2, bn_g[0].reshape(1, D), bn_b[0].reshape(1, D),
                        conv_W[0])
    acc = sc_edges(hs, src2d, dst2d)
    hs_prev = hs
    x_cur = x
    for layer in (1, 2):
        x_cur, hs = tc_mid(x_cur, acc, hs_prev,
                           dinv, conv_b[layer - 1].reshape(1, D),
                           bn_g[layer].reshape(1, D),
                           bn_b[layer].reshape(1, D), conv_W[layer])
        acc = sc_edges(hs, src2d, dst2d)
        hs_prev = hs

    wc_pad = jnp.zeros((D, G), f32).at[:, :C].set(Wc)
    bc_pad = jnp.full((G,), -1e30, f32).at[:C].set(bc)
    out = tc_head(x_cur, acc, hs_prev, dinv, conv_b[2].reshape(1, D),
                  batch.reshape(1, N), Wf, bf.reshape(1, D),
                  gf.reshape(1, D), bbf.reshape(1, D), gh.reshape(1, D),
                  bh.reshape(1, D), wc_pad, bc_pad.reshape(1, G))
    return out[:, :C]
